# Initial kernel scaffold; baseline (speedup 1.0000x reference)
import jax
import jax.numpy as jnp
from jax.experimental import pallas as pl


def kernel(channel_llrs, edge_index, alpha):
    raise NotImplementedError("write your pallas kernel here")


# speedup vs baseline: 6.8642x; 6.8642x over previous
import functools

import jax
import jax.numpy as jnp
from jax import lax
from jax.experimental import pallas as pl
from jax.experimental.pallas import tpu as pltpu
from jax.experimental.pallas import tpu_sc as plsc

F32 = jnp.float32

NV = 50000
NCK = 25000
NE = 800000
B = 16
ITERS = 5

NC = 2
NS = 16
NW = NC * NS
CHUNK = 128
CPT = 196
EPAD = NW * CPT * CHUNK
EROWS = EPAD * B // 1024
VB = 7

S_CHK = 25024
S_VAR = 50048

_mesh = plsc.VectorSubcoreMesh(
    core_axis_name="c", subcore_axis_name="s", num_cores=NC, num_subcores=NS)
_sc_params = pltpu.CompilerParams(use_tc_tiling_on_sc=False)


def _wid():
    return lax.axis_index("s") * NC + lax.axis_index("c")


def _scatter_body(nseg, vals, ids, zeros, out, ids_v, vals_v, table):
    rs = nseg // NS
    c = lax.axis_index("c")
    s = lax.axis_index("s")
    w = _wid()
    pltpu.sync_copy(zeros.at[s], table.at[pl.ds(s * rs, rs)])
    plsc.subcore_barrier()
    pltpu.sync_copy(ids.at[w], ids_v)

    def outer(jo, _):
        pltpu.sync_copy(vals.at[w, jo], vals_v)
        def inner(k, _):
            j = jo * VB + k
            pltpu.sync_copy(vals_v.at[k], table.at[ids_v.at[j]], add=True)
            return 0
        return lax.fori_loop(0, VB, inner, 0)

    lax.fori_loop(0, CPT // VB, outer, 0)
    plsc.subcore_barrier()
    pltpu.sync_copy(table.at[pl.ds(s * rs, rs)], out.at[c, s])


def _make_scatter(nseg):
    rs = nseg // NS
    return functools.partial(
        pl.kernel,
        out_type=jax.ShapeDtypeStruct((NC, NS, rs, B), F32),
        mesh=_mesh,
        compiler_params=_sc_params,
        scratch_types=[
            pltpu.VMEM((CPT, CHUNK), jnp.int32),
            pltpu.VMEM((VB, CHUNK, B), F32),
            pltpu.VMEM_SHARED((nseg, B), F32),
        ],
    )(functools.partial(_scatter_body, nseg))


_scatter_chk = _make_scatter(S_CHK)
_scatter_var = _make_scatter(S_VAR)


def _gather_body(table, ids, out, ids_v, rows_v, sem):
    w = _wid()
    pltpu.sync_copy(ids.at[w], ids_v)

    def outer(jo, _):
        def inner(k, _):
            j = jo * VB + k
            pltpu.async_copy(table.at[ids_v.at[j]], rows_v.at[k], sem).wait()
            return 0
        lax.fori_loop(0, VB, inner, 0)
        pltpu.sync_copy(rows_v, out.at[w, jo])
        return 0

    lax.fori_loop(0, CPT // VB, outer, 0)


_gather = pl.kernel(
    _gather_body,
    out_type=jax.ShapeDtypeStruct((NW, CPT // VB, VB, CHUNK, B), F32),
    mesh=_mesh,
    compiler_params=_sc_params,
    scratch_types=[
        pltpu.VMEM((CPT, CHUNK), jnp.int32),
        pltpu.VMEM((VB, CHUNK, B), F32),
        pltpu.SemaphoreType.DMA,
    ],
)


_RB = 448
_EW_GRID = EROWS // _RB
_ew_spec = pl.BlockSpec((_RB, 1024), lambda i: (i, 0))


def _e1_body(v2c_ref, lm_ref, ng_ref):
    t = jnp.tanh(v2c_ref[...] * 0.5)
    mag = jnp.clip(jnp.abs(t), 1e-7, 0.999999)
    lm_ref[...] = jnp.log(mag)
    ng_ref[...] = jnp.where(t < 0.0, 1.0, 0.0).astype(F32)


def _e1(v2c):
    return pl.pallas_call(
        _e1_body,
        grid=(_EW_GRID,),
        in_specs=[_ew_spec],
        out_specs=[_ew_spec, _ew_spec],
        out_shape=[jax.ShapeDtypeStruct((EROWS, 1024), F32)] * 2,
    )(v2c)


def _e2_body(alpha_ref, v2c_ref, gl_ref, gn_ref, c2v_ref):
    t = jnp.tanh(v2c_ref[...] * 0.5)
    mag = jnp.clip(jnp.abs(t), 1e-7, 0.999999)
    lm = jnp.log(mag)
    ng = jnp.where(t < 0.0, 1.0, 0.0).astype(F32)
    loo_log = gl_ref[...] - lm
    loo_neg = gn_ref[...] - ng
    sign = 1.0 - 2.0 * jnp.mod(loo_neg, 2.0)
    prod = jnp.clip(sign * jnp.exp(loo_log), -0.999, 0.999)
    c2v_ref[...] = alpha_ref[0, 0] * jnp.log((1.0 + prod) / (1.0 - prod))


def _e2(alpha, v2c, gl, gn):
    return pl.pallas_call(
        _e2_body,
        grid=(_EW_GRID,),
        in_specs=[
            pl.BlockSpec((1, 1), lambda i: (0, 0), memory_space=pltpu.SMEM),
            _ew_spec, _ew_spec, _ew_spec,
        ],
        out_specs=_ew_spec,
        out_shape=jax.ShapeDtypeStruct((EROWS, 1024), F32),
    )(alpha.reshape(1, 1), v2c, gl, gn)


def _e3_body(ch_ref, g_ref, c2v_ref, out_ref):
    out_ref[...] = ch_ref[...] + g_ref[...] - c2v_ref[...]


def _e3(ch, g, c2v):
    return pl.pallas_call(
        _e3_body,
        grid=(_EW_GRID,),
        in_specs=[_ew_spec] * 3,
        out_specs=_ew_spec,
        out_shape=jax.ShapeDtypeStruct((EROWS, 1024), F32),
    )(ch, g, c2v)


def _combine_body(p_ref, out_ref):
    out_ref[...] = p_ref[0] + p_ref[1]


def _combine(partials, nseg):
    rows = nseg * B // 1024
    p = partials.reshape(2, rows, 1024)
    return pl.pallas_call(
        _combine_body,
        out_shape=jax.ShapeDtypeStruct((rows, 1024), F32),
    )(p)


def _final_body(llr_ref, tab_ref, out_ref):
    out_ref[...] = llr_ref[...] + tab_ref[...]


def _final(llr_flat, tab_flat):
    rows = S_VAR * B // 1024
    return pl.pallas_call(
        _final_body,
        out_shape=jax.ShapeDtypeStruct((rows, 1024), F32),
    )(llr_flat, tab_flat)


def kernel(channel_llrs, edge_index, alpha):
    ids32 = edge_index.astype(jnp.int32)
    pad = EPAD - NE
    var_ids = jnp.concatenate(
        [ids32[0], jnp.full((pad,), NV, jnp.int32)]).reshape(NW, CPT, CHUNK)
    chk_ids = jnp.concatenate(
        [ids32[1], jnp.full((pad,), NCK, jnp.int32)]).reshape(NW, CPT, CHUNK)

    llr_tab = jnp.pad(channel_llrs.astype(F32).T, ((0, S_VAR - NV), (0, 0)))
    llr_flat = llr_tab.reshape(S_VAR * B // 1024, 1024)
    z_chk = jnp.zeros((NS, S_CHK // NS, B), F32)
    z_var = jnp.zeros((NS, S_VAR // NS, B), F32)

    def rows4(flat):
        return flat.reshape(NW, CPT // VB, VB, CHUNK, B)

    def flat2(x):
        return x.reshape(EROWS, 1024)

    ch_e = flat2(_gather(llr_tab, var_ids))
    v2c = ch_e
    tab_var = None
    for _ in range(ITERS):
        lm, ng = _e1(v2c)
        p_log = _scatter_chk(rows4(lm), chk_ids, z_chk)
        p_neg = _scatter_chk(rows4(ng), chk_ids, z_chk)
        tab_log = _combine(p_log, S_CHK).reshape(S_CHK, B)
        tab_neg = _combine(p_neg, S_CHK).reshape(S_CHK, B)
        g_log = flat2(_gather(tab_log, chk_ids))
        g_neg = flat2(_gather(tab_neg, chk_ids))
        c2v = _e2(alpha.astype(F32), v2c, g_log, g_neg)
        p_c2v = _scatter_var(rows4(c2v), var_ids, z_var)
        tab_var = _combine(p_c2v, S_VAR)
        g_c2v = flat2(_gather(tab_var.reshape(S_VAR, B), var_ids))
        v2c = _e3(ch_e, g_c2v, c2v)

    final = _final(llr_flat, tab_var).reshape(S_VAR, B)
    return final[:NV].T
